# Initial kernel scaffold; baseline (speedup 1.0000x reference)
import functools

import jax
import jax.numpy as jnp
from jax import lax
from jax.experimental import pallas as pl
from jax.experimental.pallas import tpu as pltpu
from jax.experimental.pallas import tpu_sc as plsc

B = 2
S = 8192
D = 1024
NUM_POS = S + 2

NC = 2
NS = 16
CPW = S // NS
L = 16
K = 32
T = CPW // K


@functools.partial(
    pl.kernel,
    out_type=jax.ShapeDtypeStruct((B, S, D), jnp.float32),
    mesh=plsc.VectorSubcoreMesh(core_axis_name="c", subcore_axis_name="s"),
    scratch_types=[
        pltpu.VMEM((S,), jnp.int32),
        pltpu.VMEM((CPW,), jnp.int32),
        pltpu.VMEM((2, K, D), jnp.float32),
        pltpu.SemaphoreType.DMA,
        pltpu.SemaphoreType.DMA,
    ],
)
def _sc_lookup(mask_hbm, w_hbm, out_hbm, mask_v, idx_v, buf_v, sem0, sem1):
    b = lax.axis_index("c")
    s = lax.axis_index("s")

    pltpu.sync_copy(mask_hbm.at[b], mask_v)

    n_pre = s * (CPW // L)

    def pre_body(i, acc):
        return acc + mask_v[pl.ds(i * L, L)]

    acc = lax.fori_loop(0, n_pre, pre_body, jnp.zeros((L,), jnp.int32))
    base = jnp.sum(acc)

    off = s * CPW

    def loc_body(j, run):
        v = mask_v[pl.ds(off + j * L, L)]
        c = jnp.cumsum(v) + run
        idx_v[pl.ds(j * L, L)] = c * v + 1
        return run + jnp.sum(v)

    lax.fori_loop(0, CPW // L, loc_body, base)

    sems = (sem0, sem1)
    handles = [None, None]
    handles[0] = pltpu.async_copy(
        w_hbm.at[idx_v.at[pl.ds(0, K)]], buf_v.at[0], sems[0])
    for t in range(T):
        slot = t % 2
        if t + 1 < T:
            nxt = 1 - slot
            handles[nxt] = pltpu.async_copy(
                w_hbm.at[idx_v.at[pl.ds((t + 1) * K, K)]], buf_v.at[nxt],
                sems[nxt])
        handles[slot].wait()
        pltpu.sync_copy(buf_v.at[slot],
                        out_hbm.at[b, pl.ds(off + t * K, K)])


def kernel(attention_mask, past_key_values_length, weight):
    del past_key_values_length
    return _sc_lookup(attention_mask.astype(jnp.int32), weight)


# speedup vs baseline: 1.0898x; 1.0898x over previous
import functools

import jax
import jax.numpy as jnp
from jax import lax
from jax.experimental import pallas as pl
from jax.experimental.pallas import tpu as pltpu
from jax.experimental.pallas import tpu_sc as plsc

B = 2
S = 8192
D = 1024
NUM_POS = S + 2

NC = 2
NS = 16
CPW = S // NS
L = 16
K = 32
T = CPW // K


@functools.partial(
    pl.kernel,
    out_type=jax.ShapeDtypeStruct((B, S, D), jnp.float32),
    mesh=plsc.VectorSubcoreMesh(core_axis_name="c", subcore_axis_name="s"),
    compiler_params=pltpu.CompilerParams(needs_layout_passes=False),
    scratch_types=[
        pltpu.VMEM((S,), jnp.int32),
        pltpu.VMEM((CPW,), jnp.int32),
        pltpu.VMEM((2, K, D), jnp.float32),
        pltpu.SemaphoreType.DMA,
        pltpu.SemaphoreType.DMA,
    ],
)
def _sc_lookup(mask_hbm, w_hbm, out_hbm, mask_v, idx_v, buf_v, sem0, sem1):
    b = lax.axis_index("c")
    s = lax.axis_index("s")

    pltpu.sync_copy(mask_hbm.at[b], mask_v)

    n_pre = s * (CPW // L)

    def pre_body(i, acc):
        return acc + mask_v[pl.ds(i * L, L)]

    acc = lax.fori_loop(0, n_pre, pre_body, jnp.zeros((L,), jnp.int32))
    base = jnp.sum(acc)

    off = s * CPW

    def loc_body(j, run):
        v = mask_v[pl.ds(off + j * L, L)]
        c = jnp.cumsum(v) + run
        idx_v[pl.ds(j * L, L)] = c * v + 1
        return run + jnp.sum(v)

    lax.fori_loop(0, CPW // L, loc_body, base)

    sems = (sem0, sem1)
    handles = [None, None]
    handles[0] = pltpu.async_copy(
        w_hbm.at[idx_v.at[pl.ds(0, K)]], buf_v.at[0], sems[0])
    for t in range(T):
        slot = t % 2
        if t + 1 < T:
            nxt = 1 - slot
            handles[nxt] = pltpu.async_copy(
                w_hbm.at[idx_v.at[pl.ds((t + 1) * K, K)]], buf_v.at[nxt],
                sems[nxt])
        handles[slot].wait()
        pltpu.sync_copy(buf_v.at[slot],
                        out_hbm.at[b, pl.ds(off + t * K, K)])


def kernel(attention_mask, past_key_values_length, weight):
    del past_key_values_length
    return _sc_lookup(attention_mask.astype(jnp.int32), weight)


# speedup vs baseline: 1.0912x; 1.0013x over previous
import functools

import jax
import jax.numpy as jnp
from jax import lax
from jax.experimental import pallas as pl
from jax.experimental.pallas import tpu as pltpu
from jax.experimental.pallas import tpu_sc as plsc

B = 2
S = 8192
D = 1024
NUM_POS = S + 2

NC = 2
NS = 16
CPW = S // NS
L = 16
K = 32
T = CPW // K
NBUF = 3


@functools.partial(
    pl.kernel,
    out_type=jax.ShapeDtypeStruct((B, S, D), jnp.float32),
    mesh=plsc.VectorSubcoreMesh(core_axis_name="c", subcore_axis_name="s"),
    compiler_params=pltpu.CompilerParams(needs_layout_passes=False),
    scratch_types=[
        pltpu.VMEM((S,), jnp.int32),
        pltpu.VMEM((CPW,), jnp.int32),
        pltpu.VMEM((NBUF, K, D), jnp.float32),
        [pltpu.SemaphoreType.DMA] * NBUF,
        [pltpu.SemaphoreType.DMA] * NBUF,
    ],
)
def _sc_lookup(mask_hbm, w_hbm, out_hbm, mask_v, idx_v, buf_v, gsems, wsems):
    b = lax.axis_index("c")
    s = lax.axis_index("s")

    pltpu.sync_copy(mask_hbm.at[b], mask_v)

    n_pre = s * (CPW // L)

    def pre_body(i, acc):
        return acc + mask_v[pl.ds(i * L, L)]

    acc = lax.fori_loop(0, n_pre, pre_body, jnp.zeros((L,), jnp.int32))
    base = jnp.sum(acc)

    off = s * CPW

    def loc_body(j, run):
        v = mask_v[pl.ds(off + j * L, L)]
        c = jnp.cumsum(v) + run
        idx_v[pl.ds(j * L, L)] = c * v + 1
        return run + jnp.sum(v)

    lax.fori_loop(0, CPW // L, loc_body, base)

    g_handles = [None] * NBUF
    w_handles = [None] * NBUF

    def start_gather(t, slot):
        g_handles[slot] = pltpu.async_copy(
            w_hbm.at[idx_v.at[pl.ds(t * K, K)]], buf_v.at[slot], gsems[slot])

    def start_write(t, slot):
        w_handles[slot] = pltpu.async_copy(
            buf_v.at[slot], out_hbm.at[b, pl.ds(off + t * K, K)],
            wsems[slot])

    start_gather(0, 0)
    start_gather(1, 1)
    for t in range(T):
        slot = t % NBUF
        g_handles[slot].wait()
        start_write(t, slot)
        if t + 2 < T:
            s2 = (t + 2) % NBUF
            if w_handles[s2] is not None:
                w_handles[s2].wait()
                w_handles[s2] = None
            start_gather(t + 2, s2)
    for slot in range(NBUF):
        if w_handles[slot] is not None:
            w_handles[slot].wait()


def kernel(attention_mask, past_key_values_length, weight):
    del past_key_values_length
    return _sc_lookup(attention_mask.astype(jnp.int32), weight)
